# baseline (speedup 1.0000x reference)
import functools

import jax
import jax.numpy as jnp
from jax import lax
from jax.experimental import pallas as pl
from jax.experimental.pallas import tpu as pltpu
from jax.experimental.pallas import tpu_sc as plsc

B = 16384
EMB = 64
H = 128
NC = 2
NS = 16
NW = NC * NS
BPW = B // NW

BLK = 2048


def _sc_gather_body(user_table, item_table, uid, iid, u_out, v_out,
                    idx_u, idx_v, rows_u, rows_v, sem_u, sem_v):
    wid = lax.axis_index("s") * NC + lax.axis_index("c")
    base = wid * BPW
    pltpu.sync_copy(uid.at[pl.ds(base, BPW)], idx_u)
    pltpu.sync_copy(iid.at[pl.ds(base, BPW)], idx_v)
    cu = pltpu.async_copy(user_table.at[idx_u], rows_u, sem_u)
    cv = pltpu.async_copy(item_table.at[idx_v], rows_v, sem_v)
    cu.wait()
    pltpu.sync_copy(rows_u, u_out.at[pl.ds(base, BPW)])
    cv.wait()
    pltpu.sync_copy(rows_v, v_out.at[pl.ds(base, BPW)])


_sc_gather = pl.kernel(
    _sc_gather_body,
    mesh=plsc.VectorSubcoreMesh(core_axis_name="c", subcore_axis_name="s"),
    out_type=[
        jax.ShapeDtypeStruct((B, EMB), jnp.float32),
        jax.ShapeDtypeStruct((B, EMB), jnp.float32),
    ],
    scratch_types=[
        pltpu.VMEM((BPW,), jnp.int32),
        pltpu.VMEM((BPW,), jnp.int32),
        pltpu.VMEM((BPW, EMB), jnp.float32),
        pltpu.VMEM((BPW, EMB), jnp.float32),
        pltpu.SemaphoreType.DMA,
        pltpu.SemaphoreType.DMA,
    ],
    compiler_params=pltpu.CompilerParams(use_tc_tiling_on_sc=False),
)


def _tc_towers_body(u_ref, v_ref, w0u, b0u, w1u, b1u, w0i, b0i, w1i, b1i,
                    o_ref):
    def tower(x, W0, b0, W1, b1):
        h = lax.dot_general(x, W0[...], (((1,), (1,)), ((), ())),
                            preferred_element_type=jnp.float32)
        h = jnp.maximum(h + b0[...], 0.0)
        h = lax.dot_general(h, W1[...], (((1,), (1,)), ((), ())),
                            preferred_element_type=jnp.float32)
        return jnp.maximum(h + b1[...], 0.0)

    uo = tower(u_ref[...], w0u, b0u, w1u, b1u)
    vo = tower(v_ref[...], w0i, b0i, w1i, b1i)
    o_ref[...] = jnp.sum(uo * vo, axis=-1)


def _tc_towers(u_rows, v_rows, W0_u, b0_u, W1_u, b1_u, W0_i, b0_i, W1_i, b1_i):
    full = lambda shape: pl.BlockSpec(shape, lambda i: (0,) * len(shape))
    return pl.pallas_call(
        _tc_towers_body,
        grid=(B // BLK,),
        in_specs=[
            pl.BlockSpec((BLK, EMB), lambda i: (i, 0)),
            pl.BlockSpec((BLK, EMB), lambda i: (i, 0)),
            full((H, EMB)), full((1, H)),
            full((EMB, H)), full((1, EMB)),
            full((H, EMB)), full((1, H)),
            full((EMB, H)), full((1, EMB)),
        ],
        out_specs=pl.BlockSpec((BLK,), lambda i: (i,)),
        out_shape=jax.ShapeDtypeStruct((B,), jnp.float32),
    )(u_rows, v_rows, W0_u, b0_u.reshape(1, H), W1_u, b1_u.reshape(1, EMB),
      W0_i, b0_i.reshape(1, H), W1_i, b1_i.reshape(1, EMB))


def kernel(user_ids, item_ids, user_table, item_table,
           W0_u, b0_u, W1_u, b1_u, W0_i, b0_i, W1_i, b1_i):
    uid = user_ids.astype(jnp.int32)
    iid = item_ids.astype(jnp.int32)
    u_rows, v_rows = _sc_gather(user_table, item_table, uid, iid)
    return _tc_towers(u_rows, v_rows, W0_u, b0_u, W1_u, b1_u,
                      W0_i, b0_i, W1_i, b1_i)
